# Initial kernel scaffold; baseline (speedup 1.0000x reference)
import functools

import jax
import jax.numpy as jnp
from jax import lax
from jax.experimental import pallas as pl
from jax.experimental.pallas import tpu as pltpu
from jax.experimental.pallas import tpu_sc as plsc

N = 10000
DIM = 128
HID = 64
NSUP = 2
E = 160000
NT = 16
CW = 80
NCH = E // (NT * CW)
RPT = N // NT
RB = 2000


def _tc_tables(x_u, x_v, weights_u):
    def body(xu_ref, xv_ref, w_ref, g0_ref, g1_ref):
        src = pl.program_id(0)
        xb = jnp.where(src == 0, xv_ref[...], xu_ref[...])
        w = w_ref[...]
        g0_ref[...] = jnp.dot(xb, w[:, :HID], preferred_element_type=jnp.float32)
        g1_ref[...] = jnp.dot(xb, w[:, HID:], preferred_element_type=jnp.float32)

    nrb = N // RB
    return pl.pallas_call(
        body,
        grid=(2, nrb),
        in_specs=[
            pl.BlockSpec((RB, DIM), lambda s, r: (r, 0)),
            pl.BlockSpec((RB, DIM), lambda s, r: (r, 0)),
            pl.BlockSpec((DIM, DIM), lambda s, r: (0, 0)),
        ],
        out_specs=[
            pl.BlockSpec((RB, HID), lambda s, r: (s * nrb + r, 0)),
            pl.BlockSpec((RB, HID), lambda s, r: (s * nrb + r, 0)),
        ],
        out_shape=[jax.ShapeDtypeStruct((2 * N, HID), jnp.float32)] * 2,
    )(x_u, x_v, weights_u)


def _sc_body(g0_hbm, g1_hbm, s_hbm, d_hbm, v_hbm, z0_hbm, z_hbm,
             acc0, acc1, srcb, dstb, v80, rows, stage, sem):
    c = lax.axis_index("c")
    s = lax.axis_index("s")

    pltpu.sync_copy(z0_hbm, stage)
    pltpu.sync_copy(stage, acc0.at[pl.ds(RPT * s, RPT)])
    pltpu.sync_copy(stage, acc1.at[pl.ds(RPT * s, RPT)])
    plsc.subcore_barrier()

    for i in range(NSUP):
        acc = acc0 if i == 0 else acc1
        g_hbm = g0_hbm if i == 0 else g1_hbm
        pltpu.sync_copy(s_hbm.at[c, i, s], srcb)
        pltpu.sync_copy(d_hbm.at[c, i, s], dstb)

        def chunk_body(k, carry, acc=acc, g_hbm=g_hbm, i=i):
            pltpu.sync_copy(v_hbm.at[i, s, k], v80)
            pltpu.async_copy(g_hbm.at[srcb.at[k]], rows, sem).wait()
            for g in range(CW // 16):
                v16 = v80[pl.ds(g * 16, 16)]
                for j in range(16):
                    b = jnp.take_along_axis(
                        v16, jnp.full((16,), j, jnp.int32), axis=0,
                        mode="promise_in_bounds")
                    e = g * 16 + j
                    for f in range(HID // 16):
                        sl = pl.ds(f * 16, 16)
                        rows[e, sl] = rows[e, sl] * b
            pltpu.sync_copy(rows, acc.at[dstb.at[k]], add=True)
            return carry

        lax.fori_loop(0, NCH, chunk_body, 0)

    plsc.subcore_barrier()

    for i in range(NSUP):
        acc = acc0 if i == 0 else acc1
        pltpu.sync_copy(acc.at[pl.ds(RPT * s, RPT)], stage)
        pltpu.sync_copy(stage, z_hbm.at[c, i, pl.ds(RPT * s, RPT)])


def _sc_segsum(g0, g1, S, D, V, z0):
    mesh = plsc.VectorSubcoreMesh(core_axis_name="c", subcore_axis_name="s")
    return pl.kernel(
        _sc_body,
        out_type=jax.ShapeDtypeStruct((2, NSUP, N, HID), jnp.float32),
        mesh=mesh,
        scratch_types=[
            pltpu.VMEM_SHARED((N, HID), jnp.float32),
            pltpu.VMEM_SHARED((N, HID), jnp.float32),
            pltpu.VMEM((NCH, CW), jnp.int32),
            pltpu.VMEM((NCH, CW), jnp.int32),
            pltpu.VMEM((CW,), jnp.float32),
            pltpu.VMEM((CW, HID), jnp.float32),
            pltpu.VMEM((RPT, HID), jnp.float32),
            pltpu.SemaphoreType.DMA,
        ],
    )(g0, g1, S, D, V, z0)


def _tc_finish(zraw):
    def body(zin_ref, zout_ref):
        zb = zin_ref[...]
        z = jnp.concatenate([zb[:, 0], zb[:, 1]], axis=-1)
        zout_ref[...] = jnp.maximum(z, 0.0)

    nrb = N // RB
    return pl.pallas_call(
        body,
        grid=(2, nrb),
        in_specs=[
            pl.BlockSpec((1, NSUP, RB, HID), lambda c, r: (c, 0, r, 0)),
        ],
        out_specs=pl.BlockSpec((1, RB, DIM), lambda c, r: (c, r, 0)),
        out_shape=jax.ShapeDtypeStruct((2, N, DIM), jnp.float32),
    )(zraw)


def kernel(x_u, x_v, support_rows, support_cols, support_vals, weights_u):
    g0, g1 = _tc_tables(x_u, x_v, weights_u)
    S = jnp.stack([support_cols, support_rows + N]).reshape(2, NSUP, NT, NCH, CW)
    D = jnp.stack([support_rows, support_cols]).reshape(2, NSUP, NT, NCH, CW)
    V = support_vals.reshape(NSUP, NT, NCH, CW)
    z0 = jnp.zeros((RPT, HID), jnp.float32)
    zraw = _sc_segsum(g0, g1, S, D, V, z0)
    z = _tc_finish(zraw)
    return z[0], z[1]


# speedup vs baseline: 5.3954x; 5.3954x over previous
import functools

import jax
import jax.numpy as jnp
from jax import lax
from jax.experimental import pallas as pl
from jax.experimental.pallas import tpu as pltpu
from jax.experimental.pallas import tpu_sc as plsc

N = 10000
DIM = 128
HID = 64
NSUP = 2
E = 160000
NT = 16
CW = 80
NCH = E // (NT * CW)
NPAD = 10240
RPT = NPAD // NT
STG = 320
RB = 2000


def _tc_tables(x_u, x_v, weights_u):
    def body(xu_ref, xv_ref, w_ref, g0_ref, g1_ref):
        src = pl.program_id(0)
        xb = jnp.where(src == 0, xv_ref[...], xu_ref[...])
        w = w_ref[...]
        g0_ref[...] = jnp.dot(xb, w[:, :HID], preferred_element_type=jnp.float32)
        g1_ref[...] = jnp.dot(xb, w[:, HID:], preferred_element_type=jnp.float32)

    nrb = N // RB
    return pl.pallas_call(
        body,
        grid=(2, nrb),
        in_specs=[
            pl.BlockSpec((RB, DIM), lambda s, r: (r, 0)),
            pl.BlockSpec((RB, DIM), lambda s, r: (r, 0)),
            pl.BlockSpec((DIM, DIM), lambda s, r: (0, 0)),
        ],
        out_specs=[
            pl.BlockSpec((RB, HID), lambda s, r: (s * nrb + r, 0)),
            pl.BlockSpec((RB, HID), lambda s, r: (s * nrb + r, 0)),
        ],
        out_shape=[jax.ShapeDtypeStruct((2 * N, HID), jnp.float32)] * 2,
    )(x_u, x_v, weights_u)


def _sc_body(g0_hbm, g1_hbm, s_hbm, d_hbm, v_hbm, z0_hbm, z_hbm,
             acc0, acc1, srcb, dstb, v80, rows, stage, sem):
    c = lax.axis_index("c")
    s = lax.axis_index("s")

    pltpu.sync_copy(z0_hbm, stage)
    for r in range(RPT // STG):
        pltpu.sync_copy(stage, acc0.at[pl.ds(RPT * s + STG * r, STG)])
        pltpu.sync_copy(stage, acc1.at[pl.ds(RPT * s + STG * r, STG)])
    plsc.subcore_barrier()

    for i in range(NSUP):
        acc = acc0 if i == 0 else acc1
        g_hbm = g0_hbm if i == 0 else g1_hbm
        pltpu.sync_copy(s_hbm.at[c, i, s], srcb)
        pltpu.sync_copy(d_hbm.at[c, i, s], dstb)

        def chunk_body(k, carry, acc=acc, g_hbm=g_hbm, i=i):
            pltpu.sync_copy(v_hbm.at[i, s, k], v80)
            pltpu.async_copy(g_hbm.at[srcb.at[k]], rows, sem).wait()
            for g in range(CW // 16):
                v16 = v80[pl.ds(g * 16, 16)]
                for j in range(16):
                    b = jnp.take_along_axis(
                        v16, jnp.full((16,), j, jnp.int32), axis=0,
                        mode="promise_in_bounds")
                    e = g * 16 + j
                    for f in range(HID // 16):
                        sl = pl.ds(f * 16, 16)
                        rows[e, sl] = rows[e, sl] * b
            pltpu.sync_copy(rows, acc.at[dstb.at[k]], add=True)
            return carry

        lax.fori_loop(0, NCH, chunk_body, 0)

    plsc.subcore_barrier()

    for i in range(NSUP):
        acc = acc0 if i == 0 else acc1
        for r in range(RPT // STG):
            pltpu.sync_copy(acc.at[pl.ds(RPT * s + STG * r, STG)], stage)
            pltpu.sync_copy(stage, z_hbm.at[c, i, pl.ds(RPT * s + STG * r, STG)])


def _sc_segsum(g0, g1, S, D, V, z0):
    mesh = plsc.VectorSubcoreMesh(core_axis_name="c", subcore_axis_name="s")
    return pl.kernel(
        _sc_body,
        out_type=jax.ShapeDtypeStruct((2, NSUP, NPAD, HID), jnp.float32),
        mesh=mesh,
        scratch_types=[
            pltpu.VMEM_SHARED((NPAD, HID), jnp.float32),
            pltpu.VMEM_SHARED((NPAD, HID), jnp.float32),
            pltpu.VMEM((NCH, CW), jnp.int32),
            pltpu.VMEM((NCH, CW), jnp.int32),
            pltpu.VMEM((CW,), jnp.float32),
            pltpu.VMEM((CW, HID), jnp.float32),
            pltpu.VMEM((STG, HID), jnp.float32),
            pltpu.SemaphoreType.DMA,
        ],
        compiler_params=pltpu.CompilerParams(use_tc_tiling_on_sc=False),
    )(g0, g1, S, D, V, z0)


def _tc_finish(zraw):
    def body(zin_ref, zout_ref):
        zb = zin_ref[...]
        z = jnp.concatenate([zb[:, 0], zb[:, 1]], axis=-1)
        zout_ref[...] = jnp.maximum(z, 0.0)

    nrb = N // RB
    return pl.pallas_call(
        body,
        grid=(2, nrb),
        in_specs=[
            pl.BlockSpec((1, NSUP, RB, HID), lambda c, r: (c, 0, r, 0)),
        ],
        out_specs=pl.BlockSpec((1, RB, DIM), lambda c, r: (c, r, 0)),
        out_shape=jax.ShapeDtypeStruct((2, N, DIM), jnp.float32),
    )(zraw)


def kernel(x_u, x_v, support_rows, support_cols, support_vals, weights_u):
    g0, g1 = _tc_tables(x_u, x_v, weights_u)
    S = jnp.stack([support_cols, support_rows + N]).reshape(2, NSUP, NT, NCH, CW)
    D = jnp.stack([support_rows, support_cols]).reshape(2, NSUP, NT, NCH, CW)
    V = support_vals.reshape(NSUP, NT, NCH, CW)
    z0 = jnp.zeros((STG, HID), jnp.float32)
    zraw = _sc_segsum(g0, g1, S, D, V, z0)
    z = _tc_finish(zraw)
    return z[0], z[1]


# speedup vs baseline: 8.6648x; 1.6060x over previous
import functools

import jax
import jax.numpy as jnp
from jax import lax
from jax.experimental import pallas as pl
from jax.experimental.pallas import tpu as pltpu
from jax.experimental.pallas import tpu_sc as plsc

N = 10000
DIM = 128
HID = 64
NSUP = 2
E = 160000
NT = 16
CW = 80
NCH = E // (NT * CW)
NPAD = 10240
RPT = NPAD // NT
STG = 160
RB = 2000


def _tc_tables(x_u, x_v, weights_u):
    def body(xu_ref, xv_ref, w_ref, g0_ref, g1_ref):
        src = pl.program_id(0)
        xb = jnp.where(src == 0, xv_ref[...], xu_ref[...])
        w = w_ref[...]
        g0_ref[...] = jnp.dot(xb, w[:, :HID], preferred_element_type=jnp.float32)
        g1_ref[...] = jnp.dot(xb, w[:, HID:], preferred_element_type=jnp.float32)

    nrb = N // RB
    return pl.pallas_call(
        body,
        grid=(2, nrb),
        in_specs=[
            pl.BlockSpec((RB, DIM), lambda s, r: (r, 0)),
            pl.BlockSpec((RB, DIM), lambda s, r: (r, 0)),
            pl.BlockSpec((DIM, DIM), lambda s, r: (0, 0)),
        ],
        out_specs=[
            pl.BlockSpec((RB, HID), lambda s, r: (s * nrb + r, 0)),
            pl.BlockSpec((RB, HID), lambda s, r: (s * nrb + r, 0)),
        ],
        out_shape=[jax.ShapeDtypeStruct((2 * N, HID), jnp.float32)] * 2,
    )(x_u, x_v, weights_u)


def _scale_rows(rows2, v2, b):
    for g in range(CW // 16):
        v16 = v2[b, pl.ds(g * 16, 16)]
        for j in range(16):
            bc = jnp.take_along_axis(
                v16, jnp.full((16,), j, jnp.int32), axis=0,
                mode="promise_in_bounds")
            e = g * 16 + j
            for f in range(HID // 16):
                sl = pl.ds(f * 16, 16)
                rows2[b, e, sl] = rows2[b, e, sl] * bc


def _sc_body(g0_hbm, g1_hbm, s_hbm, d_hbm, v_hbm, z0_hbm, z_hbm,
             acc0, acc1, srcb, dstb, v2, rows2, stage, sem0, sem1):
    c = lax.axis_index("c")
    s = lax.axis_index("s")
    sems = (sem0, sem1)

    pltpu.sync_copy(z0_hbm, stage)
    for r in range(RPT // STG):
        pltpu.sync_copy(stage, acc0.at[pl.ds(RPT * s + STG * r, STG)])
        pltpu.sync_copy(stage, acc1.at[pl.ds(RPT * s + STG * r, STG)])
    plsc.subcore_barrier()

    for i in range(NSUP):
        acc = acc0 if i == 0 else acc1
        g_hbm = g0_hbm if i == 0 else g1_hbm
        pltpu.sync_copy(s_hbm.at[c, i, s], srcb)
        pltpu.sync_copy(d_hbm.at[c, i, s], dstb)

        def fetch(k, b, g_hbm=g_hbm, i=i):
            pltpu.async_copy(v_hbm.at[i, s, k], v2.at[b], sems[b])
            pltpu.async_copy(g_hbm.at[srcb.at[k]], rows2.at[b], sems[b])

        def wait_fetch(b, g_hbm=g_hbm, i=i):
            pltpu.make_async_copy(v_hbm.at[i, s, 0], v2.at[b], sems[b]).wait()
            pltpu.make_async_copy(
                g_hbm.at[srcb.at[0]], rows2.at[b], sems[b]).wait()

        def do_chunk(k, b, acc=acc):
            _scale_rows(rows2, v2, b)
            pltpu.sync_copy(rows2.at[b], acc.at[dstb.at[k]], add=True)

        fetch(0, 0)

        def chunk_body(k2, carry, fetch=fetch, wait_fetch=wait_fetch,
                       do_chunk=do_chunk):
            k = 2 * k2
            wait_fetch(0)
            fetch(k + 1, 1)
            do_chunk(k, 0)
            wait_fetch(1)
            fetch(k + 2, 0)
            do_chunk(k + 1, 1)
            return carry

        lax.fori_loop(0, (NCH - 1) // 2, chunk_body, 0)
        wait_fetch(0)
        do_chunk(NCH - 1, 0)

    plsc.subcore_barrier()

    for i in range(NSUP):
        acc = acc0 if i == 0 else acc1
        for r in range(RPT // STG):
            pltpu.sync_copy(acc.at[pl.ds(RPT * s + STG * r, STG)], stage)
            pltpu.sync_copy(stage, z_hbm.at[c, i, pl.ds(RPT * s + STG * r, STG)])


def _sc_segsum(g0, g1, S, D, V, z0):
    mesh = plsc.VectorSubcoreMesh(core_axis_name="c", subcore_axis_name="s")
    return pl.kernel(
        _sc_body,
        out_type=jax.ShapeDtypeStruct((2, NSUP, NPAD, HID), jnp.float32),
        mesh=mesh,
        scratch_types=[
            pltpu.VMEM_SHARED((NPAD, HID), jnp.float32),
            pltpu.VMEM_SHARED((NPAD, HID), jnp.float32),
            pltpu.VMEM((NCH, CW), jnp.int32),
            pltpu.VMEM((NCH, CW), jnp.int32),
            pltpu.VMEM((2, CW), jnp.float32),
            pltpu.VMEM((2, CW, HID), jnp.float32),
            pltpu.VMEM((STG, HID), jnp.float32),
            pltpu.SemaphoreType.DMA,
            pltpu.SemaphoreType.DMA,
        ],
        compiler_params=pltpu.CompilerParams(use_tc_tiling_on_sc=False),
    )(g0, g1, S, D, V, z0)


def _tc_finish(zraw):
    def body(zin_ref, zout_ref):
        zb = zin_ref[...]
        z = jnp.concatenate([zb[:, 0], zb[:, 1]], axis=-1)
        zout_ref[...] = jnp.maximum(z, 0.0)

    nrb = N // RB
    return pl.pallas_call(
        body,
        grid=(2, nrb),
        in_specs=[
            pl.BlockSpec((1, NSUP, RB, HID), lambda c, r: (c, 0, r, 0)),
        ],
        out_specs=pl.BlockSpec((1, RB, DIM), lambda c, r: (c, r, 0)),
        out_shape=jax.ShapeDtypeStruct((2, N, DIM), jnp.float32),
    )(zraw)


def kernel(x_u, x_v, support_rows, support_cols, support_vals, weights_u):
    g0, g1 = _tc_tables(x_u, x_v, weights_u)
    S = jnp.stack([support_cols, support_rows + N]).reshape(2, NSUP, NT, NCH, CW)
    D = jnp.stack([support_rows, support_cols]).reshape(2, NSUP, NT, NCH, CW)
    V = support_vals.reshape(NSUP, NT, NCH, CW)
    z0 = jnp.zeros((STG, HID), jnp.float32)
    zraw = _sc_segsum(g0, g1, S, D, V, z0)
    z = _tc_finish(zraw)
    return z[0], z[1]


# speedup vs baseline: 10.8963x; 1.2575x over previous
import functools

import jax
import jax.numpy as jnp
from jax import lax
from jax.experimental import pallas as pl
from jax.experimental.pallas import tpu as pltpu
from jax.experimental.pallas import tpu_sc as plsc

N = 10000
DIM = 128
HID = 64
NSUP = 2
E = 160000
NT = 16
CW = 80
NCH = E // (NT * CW)
NPAD = 10240
RPT = NPAD // NT
STG = 160
RB = 2000


def _tc_tables(x_u, x_v, weights_u):
    def body(xu_ref, xv_ref, w_ref, g0_ref, g1_ref):
        src = pl.program_id(0)
        xb = jnp.where(src == 0, xv_ref[...], xu_ref[...])
        w = w_ref[...]
        g0_ref[...] = jnp.dot(xb, w[:, :HID], preferred_element_type=jnp.float32)
        g1_ref[...] = jnp.dot(xb, w[:, HID:], preferred_element_type=jnp.float32)

    nrb = N // RB
    return pl.pallas_call(
        body,
        grid=(2, nrb),
        in_specs=[
            pl.BlockSpec((RB, DIM), lambda s, r: (r, 0)),
            pl.BlockSpec((RB, DIM), lambda s, r: (r, 0)),
            pl.BlockSpec((DIM, DIM), lambda s, r: (0, 0)),
        ],
        out_specs=[
            pl.BlockSpec((RB, HID), lambda s, r: (s * nrb + r, 0)),
            pl.BlockSpec((RB, HID), lambda s, r: (s * nrb + r, 0)),
        ],
        out_shape=[jax.ShapeDtypeStruct((2 * N, HID), jnp.float32)] * 2,
    )(x_u, x_v, weights_u)


NSLOT = 3


def _scale_rows(rows3, v3, b):
    for g in range(CW // 16):
        v16 = v3[b, pl.ds(g * 16, 16)]
        for j in range(16):
            bc = jnp.take_along_axis(
                v16, jnp.full((16,), j, jnp.int32), axis=0,
                mode="promise_in_bounds")
            e = g * 16 + j
            for f in range(HID // 16):
                sl = pl.ds(f * 16, 16)
                rows3[b, e, sl] = rows3[b, e, sl] * bc


def _sc_body(g0_hbm, g1_hbm, s_hbm, d_hbm, v_hbm, z0_hbm, z_hbm,
             acc0, acc1, srcb, dstb, v3, rows3, stage,
             semg0, semg1, semg2, sems0, sems1, sems2):
    c = lax.axis_index("c")
    s = lax.axis_index("s")
    semg = (semg0, semg1, semg2)
    sems = (sems0, sems1, sems2)

    pltpu.sync_copy(z0_hbm, stage)
    for r in range(RPT // STG):
        pltpu.sync_copy(stage, acc0.at[pl.ds(RPT * s + STG * r, STG)])
        pltpu.sync_copy(stage, acc1.at[pl.ds(RPT * s + STG * r, STG)])
    plsc.subcore_barrier()

    for i in range(NSUP):
        acc = acc0 if i == 0 else acc1
        g_hbm = g0_hbm if i == 0 else g1_hbm
        pltpu.sync_copy(s_hbm.at[c, i, s], srcb)
        pltpu.sync_copy(d_hbm.at[c, i, s], dstb)

        def fetch(k, b, g_hbm=g_hbm, i=i):
            pltpu.async_copy(v_hbm.at[i, s, k], v3.at[b], semg[b])
            pltpu.async_copy(g_hbm.at[srcb.at[k]], rows3.at[b], semg[b])

        def wait_fetch(b, g_hbm=g_hbm, i=i):
            pltpu.make_async_copy(v_hbm.at[i, s, 0], v3.at[b], semg[b]).wait()
            pltpu.make_async_copy(
                g_hbm.at[srcb.at[0]], rows3.at[b], semg[b]).wait()

        def scat_start(k, b, acc=acc):
            pltpu.async_copy(rows3.at[b], acc.at[dstb.at[k]], sems[b],
                             add=True)

        def scat_wait(b, acc=acc):
            pltpu.make_async_copy(
                rows3.at[b], acc.at[dstb.at[0]], sems[b]).wait()

        def process(x, j, prefetch, guard_first=False):
            wait_fetch(j)
            _scale_rows(rows3, v3, j)
            scat_start(x, j)
            j2 = (j + 2) % NSLOT
            if guard_first:
                @pl.when(x > 0)
                def _():
                    scat_wait(j2)
            else:
                scat_wait(j2)
            if prefetch:
                fetch(x + 2, j2)

        fetch(0, 0)
        fetch(1, 1)

        def chunk_body(k3, carry, process=process):
            x = 3 * k3
            process(x, 0, True, guard_first=True)
            process(x + 1, 1, True)
            process(x + 2, 2, True)
            return carry

        nfull = (NCH - 2) // 3
        lax.fori_loop(0, nfull, chunk_body, 0)
        process(NCH - 2, 0, False)
        process(NCH - 1, 1, False)
        scat_wait(1)

    plsc.subcore_barrier()

    for i in range(NSUP):
        acc = acc0 if i == 0 else acc1
        for r in range(RPT // STG):
            pltpu.sync_copy(acc.at[pl.ds(RPT * s + STG * r, STG)], stage)
            pltpu.sync_copy(stage, z_hbm.at[c, i, pl.ds(RPT * s + STG * r, STG)])


def _sc_segsum(g0, g1, S, D, V, z0):
    mesh = plsc.VectorSubcoreMesh(core_axis_name="c", subcore_axis_name="s")
    return pl.kernel(
        _sc_body,
        out_type=jax.ShapeDtypeStruct((2, NSUP, NPAD, HID), jnp.float32),
        mesh=mesh,
        scratch_types=[
            pltpu.VMEM_SHARED((NPAD, HID), jnp.float32),
            pltpu.VMEM_SHARED((NPAD, HID), jnp.float32),
            pltpu.VMEM((NCH, CW), jnp.int32),
            pltpu.VMEM((NCH, CW), jnp.int32),
            pltpu.VMEM((NSLOT, CW), jnp.float32),
            pltpu.VMEM((NSLOT, CW, HID), jnp.float32),
            pltpu.VMEM((STG, HID), jnp.float32),
            pltpu.SemaphoreType.DMA,
            pltpu.SemaphoreType.DMA,
            pltpu.SemaphoreType.DMA,
            pltpu.SemaphoreType.DMA,
            pltpu.SemaphoreType.DMA,
            pltpu.SemaphoreType.DMA,
        ],
        compiler_params=pltpu.CompilerParams(use_tc_tiling_on_sc=False),
    )(g0, g1, S, D, V, z0)


def _tc_finish(zraw):
    def body(zin_ref, zout_ref):
        zb = zin_ref[...]
        z = jnp.concatenate([zb[:, 0], zb[:, 1]], axis=-1)
        zout_ref[...] = jnp.maximum(z, 0.0)

    nrb = N // RB
    return pl.pallas_call(
        body,
        grid=(2, nrb),
        in_specs=[
            pl.BlockSpec((1, NSUP, RB, HID), lambda c, r: (c, 0, r, 0)),
        ],
        out_specs=pl.BlockSpec((1, RB, DIM), lambda c, r: (c, r, 0)),
        out_shape=jax.ShapeDtypeStruct((2, N, DIM), jnp.float32),
    )(zraw)


def kernel(x_u, x_v, support_rows, support_cols, support_vals, weights_u):
    g0, g1 = _tc_tables(x_u, x_v, weights_u)
    S = jnp.stack([support_cols, support_rows + N]).reshape(2, NSUP, NT, NCH, CW)
    D = jnp.stack([support_rows, support_cols]).reshape(2, NSUP, NT, NCH, CW)
    V = support_vals.reshape(NSUP, NT, NCH, CW)
    z0 = jnp.zeros((STG, HID), jnp.float32)
    zraw = _sc_segsum(g0, g1, S, D, V, z0)
    z = _tc_finish(zraw)
    return z[0], z[1]


# speedup vs baseline: 12.4447x; 1.1421x over previous
import functools

import jax
import jax.numpy as jnp
from jax import lax
from jax.experimental import pallas as pl
from jax.experimental.pallas import tpu as pltpu
from jax.experimental.pallas import tpu_sc as plsc

N = 10000
DIM = 128
HID = 64
NSUP = 2
E = 160000
NT = 16
CW = 80
NCH = E // (NT * CW)
NPAD = 10240
RPT = NPAD // NT
STG = 160
RB = 2000


def _tc_tables(x_u, x_v, weights_u):
    def body(xu_ref, xv_ref, w_ref, g0_ref, g1_ref):
        src = pl.program_id(0)
        xb = jnp.where(src == 0, xv_ref[...], xu_ref[...])
        w = w_ref[...]
        g0_ref[...] = jnp.dot(xb, w[:, :HID], preferred_element_type=jnp.float32)
        g1_ref[...] = jnp.dot(xb, w[:, HID:], preferred_element_type=jnp.float32)

    nrb = N // RB
    return pl.pallas_call(
        body,
        grid=(2, nrb),
        in_specs=[
            pl.BlockSpec((RB, DIM), lambda s, r: (r, 0)),
            pl.BlockSpec((RB, DIM), lambda s, r: (r, 0)),
            pl.BlockSpec((DIM, DIM), lambda s, r: (0, 0)),
        ],
        out_specs=[
            pl.BlockSpec((RB, HID), lambda s, r: (s * nrb + r, 0)),
            pl.BlockSpec((RB, HID), lambda s, r: (s * nrb + r, 0)),
        ],
        out_shape=[jax.ShapeDtypeStruct((2 * N, HID), jnp.float32)] * 2,
    )(x_u, x_v, weights_u)


NSLOT = 3


def _scale_rows(rows3, v3, b):
    for g in range(CW // 16):
        v16 = v3[b, pl.ds(g * 16, 16)]
        for j in range(16):
            bc = jnp.take_along_axis(
                v16, jnp.full((16,), j, jnp.int32), axis=0,
                mode="promise_in_bounds")
            e = g * 16 + j
            for f in range(HID // 16):
                sl = pl.ds(f * 16, 16)
                rows3[b, e, sl] = rows3[b, e, sl] * bc


def _sc_body(g0_hbm, g1_hbm, r_hbm, c_hbm, v_hbm, z0_hbm, zu_hbm, zv_hbm,
             acc0, acc1, srcb, dstb, v3, rows3, stage,
             semg0, semg1, semg2, sems0, sems1, sems2):
    c = lax.axis_index("c")
    s = lax.axis_index("s")
    semg = (semg0, semg1, semg2)
    sems = (sems0, sems1, sems2)

    pltpu.sync_copy(z0_hbm, stage)
    for r in range(RPT // STG):
        pltpu.sync_copy(stage, acc0.at[pl.ds(RPT * s + STG * r, STG)])
        pltpu.sync_copy(stage, acc1.at[pl.ds(RPT * s + STG * r, STG)])
    plsc.subcore_barrier()

    for i in range(NSUP):
        acc = acc0 if i == 0 else acc1
        g_hbm = g0_hbm if i == 0 else g1_hbm

        @pl.when(c == 0)
        def _():
            pltpu.sync_copy(c_hbm.at[i, s], srcb)
            pltpu.sync_copy(r_hbm.at[i, s], dstb)

        @pl.when(c == 1)
        def _():
            pltpu.sync_copy(r_hbm.at[i, s], srcb)
            pltpu.sync_copy(c_hbm.at[i, s], dstb)
            nvec = jnp.full((16,), N, jnp.int32)

            def add_off(r, carry):
                for g in range(CW // 16):
                    sl = pl.ds(g * 16, 16)
                    srcb[r, sl] = srcb[r, sl] + nvec
                return carry

            lax.fori_loop(0, NCH, add_off, 0)

        def fetch(k, b, g_hbm=g_hbm, i=i):
            pltpu.async_copy(v_hbm.at[i, s, k], v3.at[b], semg[b])
            pltpu.async_copy(g_hbm.at[srcb.at[k]], rows3.at[b], semg[b])

        def wait_fetch(b, g_hbm=g_hbm, i=i):
            pltpu.make_async_copy(v_hbm.at[i, s, 0], v3.at[b], semg[b]).wait()
            pltpu.make_async_copy(
                g_hbm.at[srcb.at[0]], rows3.at[b], semg[b]).wait()

        def scat_start(k, b, acc=acc):
            pltpu.async_copy(rows3.at[b], acc.at[dstb.at[k]], sems[b],
                             add=True)

        def scat_wait(b, acc=acc):
            pltpu.make_async_copy(
                rows3.at[b], acc.at[dstb.at[0]], sems[b]).wait()

        def process(x, j, prefetch, guard_first=False):
            wait_fetch(j)
            _scale_rows(rows3, v3, j)
            scat_start(x, j)
            j2 = (j + 2) % NSLOT
            if guard_first:
                @pl.when(x > 0)
                def _():
                    scat_wait(j2)
            else:
                scat_wait(j2)
            if prefetch:
                fetch(x + 2, j2)

        fetch(0, 0)
        fetch(1, 1)

        def chunk_body(k3, carry, process=process):
            x = 3 * k3
            process(x, 0, True, guard_first=True)
            process(x + 1, 1, True)
            process(x + 2, 2, True)
            return carry

        nfull = (NCH - 2) // 3
        lax.fori_loop(0, nfull, chunk_body, 0)
        process(NCH - 2, 0, False)
        process(NCH - 1, 1, False)
        scat_wait(1)

    plsc.subcore_barrier()

    for i in range(NSUP):
        acc = acc0 if i == 0 else acc1
        for r in range(RPT // STG):
            pltpu.sync_copy(acc.at[pl.ds(RPT * s + STG * r, STG)], stage)

            @pl.when(c == 0)
            def _(i=i, r=r):
                pltpu.sync_copy(
                    stage, zu_hbm.at[i, pl.ds(RPT * s + STG * r, STG)])

            @pl.when(c == 1)
            def _(i=i, r=r):
                pltpu.sync_copy(
                    stage, zv_hbm.at[i, pl.ds(RPT * s + STG * r, STG)])


def _sc_segsum(g0, g1, rows_r, cols_r, vals_r, z0):
    mesh = plsc.VectorSubcoreMesh(core_axis_name="c", subcore_axis_name="s")
    return pl.kernel(
        _sc_body,
        out_type=[jax.ShapeDtypeStruct((NSUP, NPAD, HID), jnp.float32)] * 2,
        mesh=mesh,
        scratch_types=[
            pltpu.VMEM_SHARED((NPAD, HID), jnp.float32),
            pltpu.VMEM_SHARED((NPAD, HID), jnp.float32),
            pltpu.VMEM((NCH, CW), jnp.int32),
            pltpu.VMEM((NCH, CW), jnp.int32),
            pltpu.VMEM((NSLOT, CW), jnp.float32),
            pltpu.VMEM((NSLOT, CW, HID), jnp.float32),
            pltpu.VMEM((STG, HID), jnp.float32),
            pltpu.SemaphoreType.DMA,
            pltpu.SemaphoreType.DMA,
            pltpu.SemaphoreType.DMA,
            pltpu.SemaphoreType.DMA,
            pltpu.SemaphoreType.DMA,
            pltpu.SemaphoreType.DMA,
        ],
        compiler_params=pltpu.CompilerParams(use_tc_tiling_on_sc=False),
    )(g0, g1, rows_r, cols_r, vals_r, z0)


def _tc_finish(zu_raw, zv_raw):
    def body(zu_ref, zv_ref, ou_ref, ov_ref):
        zu = zu_ref[...]
        zv = zv_ref[...]
        ou_ref[...] = jnp.maximum(
            jnp.concatenate([zu[0], zu[1]], axis=-1), 0.0)
        ov_ref[...] = jnp.maximum(
            jnp.concatenate([zv[0], zv[1]], axis=-1), 0.0)

    nrb = N // RB
    return pl.pallas_call(
        body,
        grid=(nrb,),
        in_specs=[
            pl.BlockSpec((NSUP, RB, HID), lambda r: (0, r, 0)),
            pl.BlockSpec((NSUP, RB, HID), lambda r: (0, r, 0)),
        ],
        out_specs=[
            pl.BlockSpec((RB, DIM), lambda r: (r, 0)),
            pl.BlockSpec((RB, DIM), lambda r: (r, 0)),
        ],
        out_shape=[jax.ShapeDtypeStruct((N, DIM), jnp.float32)] * 2,
    )(zu_raw, zv_raw)


def kernel(x_u, x_v, support_rows, support_cols, support_vals, weights_u):
    g0, g1 = _tc_tables(x_u, x_v, weights_u)
    rows_r = support_rows.reshape(NSUP, NT, NCH, CW)
    cols_r = support_cols.reshape(NSUP, NT, NCH, CW)
    vals_r = support_vals.reshape(NSUP, NT, NCH, CW)
    z0 = jnp.zeros((STG, HID), jnp.float32)
    zu_raw, zv_raw = _sc_segsum(g0, g1, rows_r, cols_r, vals_r, z0)
    return _tc_finish(zu_raw, zv_raw)


# speedup vs baseline: 13.6284x; 1.0951x over previous
import functools

import jax
import jax.numpy as jnp
from jax import lax
from jax.experimental import pallas as pl
from jax.experimental.pallas import tpu as pltpu
from jax.experimental.pallas import tpu_sc as plsc

N = 10000
DIM = 128
HID = 64
NSUP = 2
E = 160000
NT = 16
CW = 80
NCH = E // (NT * CW)
NPAD = 10240
RPT = NPAD // NT
STG = 160
RB = 2000


def _tc_tables(x_u, x_v, weights_u):
    def body(xu_ref, xv_ref, w_ref, g0_ref, g1_ref):
        src = pl.program_id(0)
        xb = jnp.where(src == 0, xv_ref[...], xu_ref[...])
        w = w_ref[...]
        g0_ref[...] = jnp.dot(xb, w[:, :HID], preferred_element_type=jnp.float32)
        g1_ref[...] = jnp.dot(xb, w[:, HID:], preferred_element_type=jnp.float32)

    nrb = N // RB
    return pl.pallas_call(
        body,
        grid=(2, nrb),
        in_specs=[
            pl.BlockSpec((RB, DIM), lambda s, r: (r, 0)),
            pl.BlockSpec((RB, DIM), lambda s, r: (r, 0)),
            pl.BlockSpec((DIM, DIM), lambda s, r: (0, 0)),
        ],
        out_specs=[
            pl.BlockSpec((RB, HID), lambda s, r: (s * nrb + r, 0)),
            pl.BlockSpec((RB, HID), lambda s, r: (s * nrb + r, 0)),
        ],
        out_shape=[jax.ShapeDtypeStruct((2 * N, HID), jnp.float32)] * 2,
    )(x_u, x_v, weights_u)


NSLOT = 3


def _scale_rows(rows3, v3, b):
    for g in range(CW // 16):
        v16 = v3[b, pl.ds(g * 16, 16)]
        for j in range(16):
            bc = jnp.take_along_axis(
                v16, jnp.full((16,), j, jnp.int32), axis=0,
                mode="promise_in_bounds")
            e = g * 16 + j
            for f in range(HID // 16):
                sl = pl.ds(f * 16, 16)
                rows3[b, e, sl] = rows3[b, e, sl] * bc


def _sc_body(g0_hbm, g1_hbm, r_hbm, c_hbm, v_hbm, z0_hbm, zu_hbm, zv_hbm,
             acc0, acc1, srcb, dstb, v3, rows3, stage,
             semg0, semg1, semg2, sems0, sems1, sems2):
    c = lax.axis_index("c")
    s = lax.axis_index("s")
    semg = (semg0, semg1, semg2)
    sems = (sems0, sems1, sems2)

    pltpu.sync_copy(z0_hbm, stage)
    for r in range(RPT // STG):
        pltpu.sync_copy(stage, acc0.at[pl.ds(RPT * s + STG * r, STG)])
        pltpu.sync_copy(stage, acc1.at[pl.ds(RPT * s + STG * r, STG)])
    plsc.subcore_barrier()

    for i in range(NSUP):
        acc = acc0 if i == 0 else acc1
        g_hbm = g0_hbm if i == 0 else g1_hbm

        @pl.when(c == 0)
        def _():
            pltpu.sync_copy(c_hbm.at[i, s], srcb)
            pltpu.sync_copy(r_hbm.at[i, s], dstb)

        @pl.when(c == 1)
        def _():
            pltpu.sync_copy(r_hbm.at[i, s], srcb)
            pltpu.sync_copy(c_hbm.at[i, s], dstb)
            nvec = jnp.full((16,), N, jnp.int32)

            def add_off(r, carry):
                for g in range(CW // 16):
                    sl = pl.ds(g * 16, 16)
                    srcb[r, sl] = srcb[r, sl] + nvec
                return carry

            lax.fori_loop(0, NCH, add_off, 0)

        def fetch(k, b, g_hbm=g_hbm, i=i):
            pltpu.async_copy(v_hbm.at[i, s, k], v3.at[b], semg[b])
            pltpu.async_copy(g_hbm.at[srcb.at[k]], rows3.at[b], semg[b])

        def wait_fetch(b, g_hbm=g_hbm, i=i):
            pltpu.make_async_copy(v_hbm.at[i, s, 0], v3.at[b], semg[b]).wait()
            pltpu.make_async_copy(
                g_hbm.at[srcb.at[0]], rows3.at[b], semg[b]).wait()

        def scat_start(k, b, acc=acc):
            pltpu.async_copy(rows3.at[b], acc.at[dstb.at[k]], sems[b],
                             add=True)

        def scat_wait(b, acc=acc):
            pltpu.make_async_copy(
                rows3.at[b], acc.at[dstb.at[0]], sems[b]).wait()

        def process(x, j, prefetch, guard_first=False):
            wait_fetch(j)
            _scale_rows(rows3, v3, j)
            scat_start(x, j)
            j2 = (j + 2) % NSLOT
            if guard_first:
                @pl.when(x > 0)
                def _():
                    scat_wait(j2)
            else:
                scat_wait(j2)
            if prefetch:
                fetch(x + 2, j2)

        fetch(0, 0)
        fetch(1, 1)

        def chunk_body(k3, carry, process=process):
            x = 3 * k3
            process(x, 0, True, guard_first=True)
            process(x + 1, 1, True)
            process(x + 2, 2, True)
            return carry

        nfull = (NCH - 2) // 3
        lax.fori_loop(0, nfull, chunk_body, 0)
        process(NCH - 2, 0, False)
        process(NCH - 1, 1, False)
        scat_wait(1)

    plsc.subcore_barrier()

    for i in range(NSUP):
        acc = acc0 if i == 0 else acc1
        for r in range(RPT // STG):
            pltpu.sync_copy(acc.at[pl.ds(RPT * s + STG * r, STG)], stage)

            @pl.when(c == 0)
            def _(i=i, r=r):
                pltpu.sync_copy(
                    stage, zu_hbm.at[pl.ds(RPT * s + STG * r, STG),
                                     pl.ds(i * HID, HID)])

            @pl.when(c == 1)
            def _(i=i, r=r):
                pltpu.sync_copy(
                    stage, zv_hbm.at[pl.ds(RPT * s + STG * r, STG),
                                     pl.ds(i * HID, HID)])


def _sc_segsum(g0, g1, rows_r, cols_r, vals_r, z0):
    mesh = plsc.VectorSubcoreMesh(core_axis_name="c", subcore_axis_name="s")
    return pl.kernel(
        _sc_body,
        out_type=[jax.ShapeDtypeStruct((NPAD, DIM), jnp.float32)] * 2,
        mesh=mesh,
        scratch_types=[
            pltpu.VMEM_SHARED((NPAD, HID), jnp.float32),
            pltpu.VMEM_SHARED((NPAD, HID), jnp.float32),
            pltpu.VMEM((NCH, CW), jnp.int32),
            pltpu.VMEM((NCH, CW), jnp.int32),
            pltpu.VMEM((NSLOT, CW), jnp.float32),
            pltpu.VMEM((NSLOT, CW, HID), jnp.float32),
            pltpu.VMEM((STG, HID), jnp.float32),
            pltpu.SemaphoreType.DMA,
            pltpu.SemaphoreType.DMA,
            pltpu.SemaphoreType.DMA,
            pltpu.SemaphoreType.DMA,
            pltpu.SemaphoreType.DMA,
            pltpu.SemaphoreType.DMA,
        ],
        compiler_params=pltpu.CompilerParams(use_tc_tiling_on_sc=False),
    )(g0, g1, rows_r, cols_r, vals_r, z0)


def _tc_finish(zu_raw, zv_raw):
    def body(zu_ref, zv_ref, ou_ref, ov_ref):
        ou_ref[...] = jnp.maximum(zu_ref[...], 0.0)
        ov_ref[...] = jnp.maximum(zv_ref[...], 0.0)

    nrb = N // RB
    return pl.pallas_call(
        body,
        grid=(nrb,),
        in_specs=[
            pl.BlockSpec((RB, DIM), lambda r: (r, 0)),
            pl.BlockSpec((RB, DIM), lambda r: (r, 0)),
        ],
        out_specs=[
            pl.BlockSpec((RB, DIM), lambda r: (r, 0)),
            pl.BlockSpec((RB, DIM), lambda r: (r, 0)),
        ],
        out_shape=[jax.ShapeDtypeStruct((N, DIM), jnp.float32)] * 2,
    )(zu_raw, zv_raw)


def kernel(x_u, x_v, support_rows, support_cols, support_vals, weights_u):
    g0, g1 = _tc_tables(x_u, x_v, weights_u)
    rows_r = support_rows.reshape(NSUP, NT, NCH, CW)
    cols_r = support_cols.reshape(NSUP, NT, NCH, CW)
    vals_r = support_vals.reshape(NSUP, NT, NCH, CW)
    z0 = jnp.zeros((STG, HID), jnp.float32)
    zu_raw, zv_raw = _sc_segsum(g0, g1, rows_r, cols_r, vals_r, z0)
    return _tc_finish(zu_raw, zv_raw)
